# Initial kernel scaffold; baseline (speedup 1.0000x reference)
import jax
import jax.numpy as jnp
from jax.experimental import pallas as pl


def kernel(node_feat, edge_index, edge_type, weight, loop_weight, bias):
    raise NotImplementedError("write your pallas kernel here")


# speedup vs baseline: 22.1774x; 22.1774x over previous
import jax
import jax.numpy as jnp
from jax import lax
from jax.experimental import pallas as pl
from jax.experimental.pallas import tpu as pltpu
from jax.experimental.pallas import tpu_sc as plsc

N_NODES = 10000
E_EDGES = 320000
D = 128
R = 8

NC = 2
NS = 16
NW = NC * NS
B = 128
E_PAD = 327680
EROWS = E_PAD // B
RPW = EROWS // NW
CHR = 4
NCH = RPW // CHR
N_PAD = 10240
ZCH = N_PAD // B
ZPT = ZCH // NS

MBLK = 1024
NMB = -(-N_NODES // MBLK)


def _relmm_body(x_ref, w_ref, o_ref):
    o_ref[0] = jnp.dot(x_ref[...], w_ref[0], preferred_element_type=jnp.float32)


def _relation_matmul(node_feat, weight):
    return pl.pallas_call(
        _relmm_body,
        grid=(R, NMB),
        in_specs=[
            pl.BlockSpec((MBLK, D), lambda r, m: (m, 0)),
            pl.BlockSpec((1, D, D), lambda r, m: (r, 0, 0)),
        ],
        out_specs=pl.BlockSpec((1, MBLK, D), lambda r, m: (r, m, 0)),
        out_shape=jax.ShapeDtypeStruct((R, N_NODES, D), jnp.float32),
    )(node_feat, weight)


def _final_body(acc_ref, deg_ref, x_ref, w_ref, b_ref, o_ref):
    loop_msg = jnp.dot(x_ref[...], w_ref[...], preferred_element_type=jnp.float32)
    agg = acc_ref[0] + acc_ref[1] + loop_msg
    deg = deg_ref[0, :, 0:1] + deg_ref[1, :, 0:1]
    norm = 1.0 / (deg + 1.0)
    o_ref[...] = jnp.maximum(agg * norm + b_ref[...], 0.0)


def _final_update(acc, deg, node_feat, loop_weight, bias):
    return pl.pallas_call(
        _final_body,
        grid=(NMB,),
        in_specs=[
            pl.BlockSpec((NC, MBLK, D), lambda m: (0, m, 0)),
            pl.BlockSpec((NC, MBLK, D), lambda m: (0, m, 0)),
            pl.BlockSpec((MBLK, D), lambda m: (m, 0)),
            pl.BlockSpec((D, D), lambda m: (0, 0)),
            pl.BlockSpec((1, D), lambda m: (0, 0)),
        ],
        out_specs=pl.BlockSpec((MBLK, D), lambda m: (m, 0)),
        out_shape=jax.ShapeDtypeStruct((N_NODES, D), jnp.float32),
    )(acc, deg, node_feat, loop_weight, bias.reshape(1, D))


def _sc_body(table, fid2, dst2, ident3, zrow_h, ones_h,
             acc_out, deg_out,
             idx_v, dst_v, zidx_v, rows_v, acc_s, sem):
    cid = lax.axis_index("c")
    sid = lax.axis_index("s")
    wid = sid * NC + cid

    pltpu.sync_copy(ident3.at[sid], zidx_v)
    pltpu.sync_copy(zrow_h, rows_v)

    for p in range(ZPT):
        pltpu.sync_copy(rows_v, acc_s.at[zidx_v.at[p]])

    plsc.subcore_barrier()

    @pl.loop(0, NCH)
    def _chunk(c):
        row0 = wid * RPW + c * CHR
        pltpu.sync_copy(fid2.at[pl.ds(row0, CHR)], idx_v)
        pltpu.sync_copy(dst2.at[pl.ds(row0, CHR)], dst_v)
        for j in range(CHR):
            pltpu.async_copy(table.at[idx_v.at[j]], rows_v, sem).wait()
            pltpu.sync_copy(rows_v, acc_s.at[dst_v.at[j]], add=True)

    plsc.subcore_barrier()

    for p in range(ZPT):
        base = (sid * ZPT + p) * B
        pltpu.async_copy(acc_s.at[zidx_v.at[p]], rows_v, sem).wait()
        pltpu.sync_copy(rows_v, acc_out.at[cid, pl.ds(base, B)])

    plsc.subcore_barrier()

    pltpu.sync_copy(zrow_h, rows_v)
    for p in range(ZPT):
        pltpu.sync_copy(rows_v, acc_s.at[zidx_v.at[p]])

    plsc.subcore_barrier()

    pltpu.sync_copy(ones_h, rows_v)

    @pl.loop(0, NCH)
    def _dchunk(c):
        row0 = wid * RPW + c * CHR
        pltpu.sync_copy(dst2.at[pl.ds(row0, CHR)], dst_v)
        for j in range(CHR):
            pltpu.sync_copy(rows_v, acc_s.at[dst_v.at[j]], add=True)

    plsc.subcore_barrier()

    for p in range(ZPT):
        base = (sid * ZPT + p) * B
        pltpu.async_copy(acc_s.at[zidx_v.at[p]], rows_v, sem).wait()
        pltpu.sync_copy(rows_v, deg_out.at[cid, pl.ds(base, B)])


def _sc_gather_scatter(table, fid2, dst2, ident3):
    zrow_h = jnp.zeros((B, D), jnp.float32)
    ones_h = jnp.ones((B, D), jnp.float32)
    mesh = plsc.VectorSubcoreMesh(core_axis_name="c", subcore_axis_name="s",
                                  num_cores=NC, num_subcores=NS)
    f = pl.kernel(
        _sc_body,
        mesh=mesh,
        out_type=[
            jax.ShapeDtypeStruct((NC, N_PAD, D), jnp.float32),
            jax.ShapeDtypeStruct((NC, N_PAD, D), jnp.float32),
        ],
        scratch_types=[
            pltpu.VMEM((CHR, B), jnp.int32),
            pltpu.VMEM((CHR, B), jnp.int32),
            pltpu.VMEM((ZPT, B), jnp.int32),
            pltpu.VMEM((B, D), jnp.float32),
            pltpu.VMEM_SHARED((N_PAD, D), jnp.float32),
            pltpu.SemaphoreType.DMA,
        ],
    )
    return f(table, fid2, dst2, ident3, zrow_h, ones_h)


def _pad2d(x, fill):
    pad = jnp.full((E_PAD - E_EDGES,), fill, jnp.int32)
    return jnp.concatenate([x.astype(jnp.int32), pad]).reshape(EROWS, B)


def kernel(node_feat, edge_index, edge_type, weight, loop_weight, bias):
    fused = (edge_type.astype(jnp.int32) * N_NODES
             + edge_index[0].astype(jnp.int32))
    npad = E_PAD - E_EDGES
    pad_idx = jnp.arange(npad, dtype=jnp.int32) % (R * N_NODES)
    pad_dst = N_NODES + (jnp.arange(npad, dtype=jnp.int32) % (N_PAD - N_NODES))
    fid2 = jnp.concatenate([fused, pad_idx]).reshape(EROWS, B)
    dst2 = jnp.concatenate([edge_index[1].astype(jnp.int32),
                            pad_dst]).reshape(EROWS, B)
    ident3 = jnp.arange(N_PAD, dtype=jnp.int32).reshape(NS, ZPT, B)

    h_all = _relation_matmul(node_feat, weight)
    table = h_all.reshape(R * N_NODES, D)
    acc, deg = _sc_gather_scatter(table, fid2, dst2, ident3)
    out = _final_update(acc, deg, node_feat, loop_weight, bias)
    return (out, edge_type)


# speedup vs baseline: 23.5457x; 1.0617x over previous
import jax
import jax.numpy as jnp
from jax import lax
from jax.experimental import pallas as pl
from jax.experimental.pallas import tpu as pltpu
from jax.experimental.pallas import tpu_sc as plsc

N_NODES = 10000
E_EDGES = 320000
D = 128
R = 8

NC = 2
NS = 16
NW = NC * NS
B = 64
E_PAD = 327680
EROWS = E_PAD // B
RPW = EROWS // NW
CHR = 8
NCH = RPW // CHR
N_PAD = 10240
ZCH = N_PAD // B
ZPT = ZCH // NS

MBLK = 1024
NMB = -(-N_NODES // MBLK)


def _relmm_body(x_ref, w_ref, o_ref):
    o_ref[0] = jnp.dot(x_ref[...], w_ref[0], preferred_element_type=jnp.float32)


def _relation_matmul(node_feat, weight):
    return pl.pallas_call(
        _relmm_body,
        grid=(R, NMB),
        in_specs=[
            pl.BlockSpec((MBLK, D), lambda r, m: (m, 0)),
            pl.BlockSpec((1, D, D), lambda r, m: (r, 0, 0)),
        ],
        out_specs=pl.BlockSpec((1, MBLK, D), lambda r, m: (r, m, 0)),
        out_shape=jax.ShapeDtypeStruct((R, N_NODES, D), jnp.float32),
    )(node_feat, weight)


def _final_body(acc_ref, deg_ref, x_ref, w_ref, b_ref, o_ref):
    loop_msg = jnp.dot(x_ref[...], w_ref[...], preferred_element_type=jnp.float32)
    agg = acc_ref[0] + acc_ref[1] + loop_msg
    deg = deg_ref[0, :, 0:1] + deg_ref[1, :, 0:1]
    norm = 1.0 / (deg + 1.0)
    o_ref[...] = jnp.maximum(agg * norm + b_ref[...], 0.0)


def _final_update(acc, deg, node_feat, loop_weight, bias):
    return pl.pallas_call(
        _final_body,
        grid=(NMB,),
        in_specs=[
            pl.BlockSpec((NC, MBLK, D), lambda m: (0, m, 0)),
            pl.BlockSpec((NC, MBLK, D), lambda m: (0, m, 0)),
            pl.BlockSpec((MBLK, D), lambda m: (m, 0)),
            pl.BlockSpec((D, D), lambda m: (0, 0)),
            pl.BlockSpec((1, D), lambda m: (0, 0)),
        ],
        out_specs=pl.BlockSpec((MBLK, D), lambda m: (m, 0)),
        out_shape=jax.ShapeDtypeStruct((N_NODES, D), jnp.float32),
    )(acc, deg, node_feat, loop_weight, bias.reshape(1, D))


def _sc_body(table, fid2, dst2, ident3, zrow_h, ones_h,
             acc_out, deg_out,
             idx_v, dst_v, zidx_v, rows_v, rows_w, acc_s, sem_a, sem_b):
    cid = lax.axis_index("c")
    sid = lax.axis_index("s")
    wid = sid * NC + cid

    pltpu.sync_copy(ident3.at[sid], zidx_v)
    pltpu.sync_copy(zrow_h, rows_v)

    for p in range(ZPT):
        pltpu.sync_copy(rows_v, acc_s.at[zidx_v.at[p]])

    plsc.subcore_barrier()

    @pl.loop(0, NCH)
    def _chunk(c):
        row0 = wid * RPW + c * CHR
        pltpu.sync_copy(fid2.at[pl.ds(row0, CHR)], idx_v)
        pltpu.sync_copy(dst2.at[pl.ds(row0, CHR)], dst_v)
        bufs = (rows_v, rows_w)
        sems = (sem_a, sem_b)
        descs = [None] * CHR
        descs[0] = pltpu.async_copy(table.at[idx_v.at[0]], bufs[0], sems[0])
        for j in range(CHR):
            if j + 1 < CHR:
                descs[j + 1] = pltpu.async_copy(
                    table.at[idx_v.at[j + 1]], bufs[(j + 1) % 2],
                    sems[(j + 1) % 2])
            descs[j].wait()
            pltpu.sync_copy(bufs[j % 2], acc_s.at[dst_v.at[j]], add=True)

    plsc.subcore_barrier()

    for p in range(ZPT):
        base = (sid * ZPT + p) * B
        pltpu.async_copy(acc_s.at[zidx_v.at[p]], rows_v, sem_a).wait()
        pltpu.sync_copy(rows_v, acc_out.at[cid, pl.ds(base, B)])

    plsc.subcore_barrier()

    pltpu.sync_copy(zrow_h, rows_v)
    for p in range(ZPT):
        pltpu.sync_copy(rows_v, acc_s.at[zidx_v.at[p]])

    plsc.subcore_barrier()

    pltpu.sync_copy(ones_h, rows_v)

    @pl.loop(0, NCH)
    def _dchunk(c):
        row0 = wid * RPW + c * CHR
        pltpu.sync_copy(dst2.at[pl.ds(row0, CHR)], dst_v)
        for j in range(CHR):
            pltpu.sync_copy(rows_v, acc_s.at[dst_v.at[j]], add=True)

    plsc.subcore_barrier()

    for p in range(ZPT):
        base = (sid * ZPT + p) * B
        pltpu.async_copy(acc_s.at[zidx_v.at[p]], rows_v, sem_a).wait()
        pltpu.sync_copy(rows_v, deg_out.at[cid, pl.ds(base, B)])


def _sc_gather_scatter(table, fid2, dst2, ident3):
    zrow_h = jnp.zeros((B, D), jnp.float32)
    ones_h = jnp.ones((B, D), jnp.float32)
    mesh = plsc.VectorSubcoreMesh(core_axis_name="c", subcore_axis_name="s",
                                  num_cores=NC, num_subcores=NS)
    f = pl.kernel(
        _sc_body,
        mesh=mesh,
        out_type=[
            jax.ShapeDtypeStruct((NC, N_PAD, D), jnp.float32),
            jax.ShapeDtypeStruct((NC, N_PAD, D), jnp.float32),
        ],
        scratch_types=[
            pltpu.VMEM((CHR, B), jnp.int32),
            pltpu.VMEM((CHR, B), jnp.int32),
            pltpu.VMEM((ZPT, B), jnp.int32),
            pltpu.VMEM((B, D), jnp.float32),
            pltpu.VMEM((B, D), jnp.float32),
            pltpu.VMEM_SHARED((N_PAD, D), jnp.float32),
            pltpu.SemaphoreType.DMA,
            pltpu.SemaphoreType.DMA,
        ],
    )
    return f(table, fid2, dst2, ident3, zrow_h, ones_h)


def _pad2d(x, fill):
    pad = jnp.full((E_PAD - E_EDGES,), fill, jnp.int32)
    return jnp.concatenate([x.astype(jnp.int32), pad]).reshape(EROWS, B)


def kernel(node_feat, edge_index, edge_type, weight, loop_weight, bias):
    fused = (edge_type.astype(jnp.int32) * N_NODES
             + edge_index[0].astype(jnp.int32))
    npad = E_PAD - E_EDGES
    pad_idx = jnp.arange(npad, dtype=jnp.int32) % (R * N_NODES)
    pad_dst = N_NODES + (jnp.arange(npad, dtype=jnp.int32) % (N_PAD - N_NODES))
    fid2 = jnp.concatenate([fused, pad_idx]).reshape(EROWS, B)
    dst2 = jnp.concatenate([edge_index[1].astype(jnp.int32),
                            pad_dst]).reshape(EROWS, B)
    ident3 = jnp.arange(N_PAD, dtype=jnp.int32).reshape(NS, ZPT, B)

    h_all = _relation_matmul(node_feat, weight)
    table = h_all.reshape(R * N_NODES, D)
    acc, deg = _sc_gather_scatter(table, fid2, dst2, ident3)
    out = _final_update(acc, deg, node_feat, loop_weight, bias)
    return (out, edge_type)
